# baseline (speedup 1.0000x reference)
import functools

import jax
import jax.numpy as jnp
from jax import lax
from jax.experimental import pallas as pl
from jax.experimental.pallas import tpu as pltpu
from jax.experimental.pallas import tpu_sc as plsc

_EMB = 18432
_B = 64
_S = 128
_TOTAL = _B * _S
_NC, _NS = 2, 16
_NW = _NC * _NS
_RPT = _TOTAL // _NW
_L = 16
_NWIN = _RPT // _L
_NBUF = 4

_mesh = plsc.VectorSubcoreMesh(core_axis_name="c", subcore_axis_name="s")


@functools.partial(
    pl.kernel,
    mesh=_mesh,
    out_type=jax.ShapeDtypeStruct((_TOTAL, 1, _EMB), jnp.float32),
    scratch_types=[
        pltpu.VMEM((_NWIN, _L), jnp.int32),
        pltpu.VMEM((_NWIN, _L), jnp.int32),
        pltpu.VMEM((_NBUF, 1, _EMB), jnp.float32),
        pltpu.SemaphoreType.DMA,
        pltpu.SemaphoreType.DMA,
        pltpu.SemaphoreType.DMA,
        pltpu.SemaphoreType.DMA,
        pltpu.SemaphoreType.DMA,
    ],
)
def _gather(table_hbm, sv_hbm, pos_hbm, out_hbm, sv_v, pos_v, buf, gsem,
            s0, s1, s2, s3):
    ssem = (s0, s1, s2, s3)
    wid = lax.axis_index("s") * _NC + lax.axis_index("c")
    pltpu.sync_copy(sv_hbm.at[wid], sv_v)
    pltpu.sync_copy(pos_hbm.at[wid], pos_v)

    def swait(b):
        pltpu.make_async_copy(buf.at[b], out_hbm.at[0], ssem[b]).wait()

    def window(w, carry):
        prev, u, c0, c1, c2, c3, w0, w1, w2, w3 = carry
        cs = [c0, c1, c2, c3]
        ws = [w0, w1, w2, w3]
        sv_win = sv_v[w, :]
        pos_win = pos_v[w, :]
        for l in range(_L):
            v = sv_win[l]
            p = pos_win[l]
            h = v != prev
            u = u + h.astype(jnp.int32)
            s = lax.rem(u - 1, _NBUF)
            for b in range(_NBUF):
                @pl.when(jnp.logical_and(h, s == b))
                def _(b=b):
                    lax.fori_loop(
                        ws[b], cs[b],
                        lambda i, cy: (swait(b), cy)[1], 0)
                    pltpu.async_copy(
                        table_hbm.at[sv_v.at[w, pl.ds(l, 1)]],
                        buf.at[b], gsem).wait()

                @pl.when(s == b)
                def _(b=b):
                    pltpu.async_copy(buf.at[b], out_hbm.at[p], ssem[b])

            for b in range(_NBUF):
                ws[b] = jnp.where(jnp.logical_and(h, s == b), cs[b], ws[b])
                cs[b] = jnp.where(s == b, cs[b] + 1, cs[b])
            prev = v
        return (prev, u, cs[0], cs[1], cs[2], cs[3],
                ws[0], ws[1], ws[2], ws[3])

    zero = jnp.int32(0)
    carry = lax.fori_loop(
        0, _NWIN, window,
        (jnp.int32(-1), zero, zero, zero, zero, zero, zero, zero, zero, zero))
    _, _, c0, c1, c2, c3, w0, w1, w2, w3 = carry
    cs = (c0, c1, c2, c3)
    ws = (w0, w1, w2, w3)
    for b in range(_NBUF):
        lax.fori_loop(ws[b], cs[b], lambda i, cy: (swait(b), cy)[1], 0)


def kernel(prefix, embedding_table):
    flat = prefix.astype(jnp.int32).reshape(_TOTAL)
    packed = jnp.sort(flat * _TOTAL + jnp.arange(_TOTAL, dtype=jnp.int32))
    sv = (packed // _TOTAL).reshape(_NW, _NWIN, _L)
    pos = (packed % _TOTAL).reshape(_NW, _NWIN, _L)
    out = _gather(embedding_table, sv, pos)
    return out.reshape(_B, _S, _EMB)
